# baseline (speedup 1.0000x reference)
import functools

import jax
import jax.numpy as jnp
from jax import lax
from jax.experimental import pallas as pl
from jax.experimental.pallas import tpu as pltpu
from jax.experimental.pallas import tpu_sc as plsc

B, S, F = 8, 1024, 64
K = 32
OUT_F = 128

NW = 32
ROWS_B = K * S
PER_W = ROWS_B // NW
CH = 128
NCH = PER_W // CH

TS = 64
ZBLK = 256


def _zz_kernel(xflat_ref, w1at_ref, w1bt_ref, b1_ref, zz_ref, y_ref):
    y = jnp.dot(xflat_ref[:], w1bt_ref[:], preferred_element_type=jnp.float32)
    y_ref[:] = y
    zz_ref[:] = (
        jnp.dot(xflat_ref[:], w1at_ref[:], preferred_element_type=jnp.float32)
        + y + b1_ref[:]
    )


def _topk_kernel(x_ref, xt_ref, idx_ref, d_ref):
    xb = x_ref[0]
    xt = xt_ref[0]
    s2 = jnp.sum(xt * xt, axis=0, keepdims=True)
    gram = jnp.dot(xb, xt, preferred_element_type=jnp.float32)
    d_ref[:] = s2 - 2.0 * gram
    colf = lax.broadcasted_iota(jnp.int32, (S, S), 1).astype(jnp.float32)
    kiota = lax.broadcasted_iota(jnp.int32, (S, K), 1)
    bigf = jnp.float32(3.0e38)
    inf = jnp.float32(3.0e38)

    def body(kk, acc):
        dv = d_ref[:]
        m = jnp.min(dv, axis=1, keepdims=True)
        cand = jnp.where(dv <= m, colf, bigf)
        aminf = jnp.min(cand, axis=1, keepdims=True)
        d_ref[:] = jnp.where(cand == aminf, inf, dv)
        return jnp.where(kiota == kk, aminf.astype(jnp.int32), acc)

    idx_ref[:] = lax.fori_loop(0, K, body, jnp.zeros((S, K), jnp.int32))


NB = 6


@functools.lru_cache(maxsize=1)
def _make_gather_sc():
    @functools.partial(
        pl.kernel,
        mesh=plsc.VectorSubcoreMesh(core_axis_name="c", subcore_axis_name="s"),
        out_type=jax.ShapeDtypeStruct((ROWS_B, OUT_F), jnp.float32),
        scratch_types=(
            [pltpu.VMEM((NCH, CH), jnp.int32)]
            + [pltpu.VMEM((CH, OUT_F), jnp.float32) for _ in range(NB)]
            + [pltpu.SemaphoreType.DMA, pltpu.SemaphoreType.DMA]
        ),
    )
    def gather_sc(idx_hbm, table_hbm, out_hbm, idx_v, *rest):
        bufs = rest[:NB]
        gsem, ssem = rest[NB:]
        cid = lax.axis_index("c")
        sid = lax.axis_index("s")
        wid = sid * 2 + cid
        base = wid * PER_W
        pltpu.sync_copy(idx_hbm.at[wid], idx_v)

        gcp = [
            pltpu.async_copy(table_hbm.at[idx_v.at[c]], bufs[c], gsem)
            for c in range(NB)
        ]
        scp = []
        for c in range(NCH):
            gcp[c].wait()
            scp.append(pltpu.async_copy(
                bufs[c % NB], out_hbm.at[pl.ds(base + c * CH, CH)], ssem))
            nxt = c + NB
            if nxt < NCH:
                scp[c].wait()
                gcp.append(pltpu.async_copy(
                    table_hbm.at[idx_v.at[nxt]], bufs[nxt % NB], gsem))
        for c in range(NCH - NB, NCH):
            scp[c].wait()

    return gather_sc


def _gather_sc(idx3, table):
    return _make_gather_sc()(idx3, table)


def _mlp_kernel(*refs):
    g_refs = refs[:B]
    zz3_ref, gam_ref, bet_ref, w2t_ref, b2_ref, out_ref = refs[B:]
    n_inv = jnp.float32(1.0 / (B * K * OUT_F))
    zzb = zz3_ref[:]
    hs = [zzb - g_refs[b][:] for b in range(B)]
    s1 = hs[0]
    for h in hs[1:]:
        s1 = s1 + h
    s1 = jnp.sum(s1, axis=(0, 2), keepdims=True)
    s2 = jnp.sum(sum(h * h for h in hs), axis=(0, 2), keepdims=True)
    mean = s1 * n_inv
    var = s2 * n_inv - mean * mean
    inv = lax.rsqrt(var + 1e-5)
    gam = gam_ref[:].reshape(1, TS, 1)
    bet = bet_ref[:].reshape(1, TS, 1)
    w2t = w2t_ref[:]
    b2v = b2_ref[:]
    h2s = []
    for h in hs:
        a1 = jnp.maximum((h - mean) * inv * gam + bet, 0.0)
        h2 = jnp.dot(a1.reshape(K * TS, OUT_F), w2t,
                     preferred_element_type=jnp.float32) + b2v
        h2s.append(h2.reshape(K, TS, OUT_F))
    s1b = jnp.sum(sum(h2s), axis=(0, 2), keepdims=True)
    s2b = jnp.sum(sum(h * h for h in h2s), axis=(0, 2), keepdims=True)
    meanb = s1b * n_inv
    varb = s2b * n_inv - meanb * meanb
    invb = lax.rsqrt(varb + 1e-5)
    for b in range(B):
        a2 = jnp.maximum((h2s[b] - meanb) * invb * gam + bet, 0.0)
        out_ref[b] = jnp.max(a2, axis=0)


def kernel(x, W1, b1, W2, b2, gamma, beta):
    xflat = x.reshape(B * S, F)
    xt = jnp.transpose(x, (0, 2, 1))
    w1at = jnp.transpose(W1[:, :F])
    w1bt = jnp.transpose(W1[:, F:])
    w2t = jnp.transpose(W2)
    b1r = b1.reshape(1, OUT_F)
    b2r = b2.reshape(1, OUT_F)
    gam2 = gamma.reshape(S, 1)
    bet2 = beta.reshape(S, 1)

    zz, y = pl.pallas_call(
        _zz_kernel,
        out_shape=(
            jax.ShapeDtypeStruct((B * S, OUT_F), jnp.float32),
            jax.ShapeDtypeStruct((B * S, OUT_F), jnp.float32),
        ),
    )(xflat, w1at, w1bt, b1r)
    zz3 = zz.reshape(ZBLK, K, OUT_F).transpose(1, 0, 2)

    gs = []
    for b in range(B):
        idx_b = pl.pallas_call(
            _topk_kernel,
            grid=(1,),
            in_specs=[
                pl.BlockSpec((1, S, F), lambda i, b=b: (b, 0, 0)),
                pl.BlockSpec((1, F, S), lambda i, b=b: (b, 0, 0)),
            ],
            out_specs=pl.BlockSpec((S, K), lambda i: (0, 0)),
            out_shape=jax.ShapeDtypeStruct((S, K), jnp.int32),
            scratch_shapes=[pltpu.VMEM((S, S), jnp.float32)],
        )(x, xt)
        idx3_b = (jnp.transpose(idx_b) + jnp.int32(b * S)).reshape(NW, NCH, CH)
        gs.append(_gather_sc(idx3_b, y).reshape(K, S, OUT_F))

    g_specs = [
        pl.BlockSpec((K, TS, OUT_F), lambda i: (0, i, 0)) for _ in range(B)
    ]
    out = pl.pallas_call(
        _mlp_kernel,
        grid=(S // TS,),
        in_specs=g_specs + [
            pl.BlockSpec((K, TS, OUT_F), lambda i: (0, i % (ZBLK // TS), 0)),
            pl.BlockSpec((TS, 1), lambda i: (i, 0)),
            pl.BlockSpec((TS, 1), lambda i: (i, 0)),
            pl.BlockSpec((OUT_F, OUT_F), lambda i: (0, 0)),
            pl.BlockSpec((1, OUT_F), lambda i: (0, 0)),
        ],
        out_specs=pl.BlockSpec((B, TS, OUT_F), lambda i: (0, i, 0)),
        out_shape=jax.ShapeDtypeStruct((B, S, OUT_F), jnp.float32),
    )(*gs, zz3, gam2, bet2, w2t, b2r)
    return out


# speedup vs baseline: 1.0366x; 1.0366x over previous
import functools

import jax
import jax.numpy as jnp
from jax import lax
from jax.experimental import pallas as pl
from jax.experimental.pallas import tpu as pltpu
from jax.experimental.pallas import tpu_sc as plsc

B, S, F = 8, 1024, 64
K = 32
OUT_F = 128

NW = 32
ROWS_B = K * S
PER_W = ROWS_B // NW
CH = 128
NCH = PER_W // CH

TS = 64


def _zz_kernel(xflat_ref, w1at_ref, w1bt_ref, b1_ref, zz_ref, y_ref):
    y = jnp.dot(xflat_ref[:], w1bt_ref[:], preferred_element_type=jnp.float32)
    y_ref[:] = y
    zz_ref[:] = (
        jnp.dot(xflat_ref[:], w1at_ref[:], preferred_element_type=jnp.float32)
        + y + b1_ref[:]
    )


def _topk_kernel(x_ref, xt_ref, idx_ref, d_ref, *, boff):
    xb = x_ref[0]
    xt = xt_ref[0]
    s2 = jnp.sum(xt * xt, axis=0, keepdims=True)
    gram = jnp.dot(xb, xt, preferred_element_type=jnp.float32)
    d0 = s2 - 2.0 * gram
    d_ref[:] = d0
    m0 = jnp.min(d0, axis=1, keepdims=True)
    colf = lax.broadcasted_iota(jnp.int32, (S, S), 1).astype(jnp.float32)
    kiota = lax.broadcasted_iota(jnp.int32, (S, K), 1)
    bigf = jnp.float32(3.0e38)
    inf = jnp.float32(3.0e38)

    def body(kk, carry):
        m, acc = carry
        dv = d_ref[:]
        cand = jnp.where(dv <= m, colf, bigf)
        aminf = jnp.min(cand, axis=1, keepdims=True)
        dv2 = jnp.where(colf == aminf, inf, dv)
        d_ref[:] = dv2
        m2 = jnp.min(dv2, axis=1, keepdims=True)
        return (m2, jnp.where(kiota == kk, aminf + boff, acc))

    _, accf = lax.fori_loop(0, K, body, (m0, jnp.zeros((S, K), jnp.float32)))
    idx_ref[:] = accf.astype(jnp.int32)


NB = 6


@functools.lru_cache(maxsize=1)
def _make_gather_sc():
    @functools.partial(
        pl.kernel,
        mesh=plsc.VectorSubcoreMesh(core_axis_name="c", subcore_axis_name="s"),
        out_type=jax.ShapeDtypeStruct((ROWS_B, OUT_F), jnp.float32),
        scratch_types=(
            [pltpu.VMEM((NCH, CH), jnp.int32)]
            + [pltpu.VMEM((CH, OUT_F), jnp.float32) for _ in range(NB)]
            + [pltpu.SemaphoreType.DMA, pltpu.SemaphoreType.DMA]
        ),
    )
    def gather_sc(idx_hbm, table_hbm, out_hbm, idx_v, *rest):
        bufs = rest[:NB]
        gsem, ssem = rest[NB:]
        cid = lax.axis_index("c")
        sid = lax.axis_index("s")
        wid = sid * 2 + cid
        base = wid * PER_W
        pltpu.sync_copy(idx_hbm.at[wid], idx_v)

        gcp = [
            pltpu.async_copy(table_hbm.at[idx_v.at[c]], bufs[c], gsem)
            for c in range(NB)
        ]
        scp = []
        for c in range(NCH):
            gcp[c].wait()
            scp.append(pltpu.async_copy(
                bufs[c % NB], out_hbm.at[pl.ds(base + c * CH, CH)], ssem))
            nxt = c + NB
            if nxt < NCH:
                scp[c].wait()
                gcp.append(pltpu.async_copy(
                    table_hbm.at[idx_v.at[nxt]], bufs[nxt % NB], gsem))
        for c in range(NCH - NB, NCH):
            scp[c].wait()

    return gather_sc


def _gather_sc(idx3, table):
    return _make_gather_sc()(idx3, table)


def _mlp_kernel(*refs):
    g_refs = refs[:B]
    zz_ref, gam_ref, bet_ref, w2t_ref, b2_ref, out_ref = refs[B:]
    n_inv = jnp.float32(1.0 / (B * K * OUT_F))
    zzb = zz_ref[:].reshape(TS, K, OUT_F)
    s1 = jnp.zeros((TS, 1, 1), jnp.float32)
    s2 = jnp.zeros((TS, 1, 1), jnp.float32)
    for b in range(B):
        h = zzb - g_refs[b][:].reshape(TS, K, OUT_F)
        s1 = s1 + jnp.sum(h, axis=(1, 2), keepdims=True)
        s2 = s2 + jnp.sum(h * h, axis=(1, 2), keepdims=True)
    mean = s1 * n_inv
    var = s2 * n_inv - mean * mean
    gam = gam_ref[:].reshape(TS, 1, 1)
    bet = bet_ref[:].reshape(TS, 1, 1)
    sc = lax.rsqrt(var + 1e-5) * gam
    sh = bet - mean * sc
    w2t = w2t_ref[:]
    b2v = b2_ref[:]
    h2s = []
    for b in range(B):
        h = zzb - g_refs[b][:].reshape(TS, K, OUT_F)
        a1 = jnp.maximum(h * sc + sh, 0.0)
        h2 = jnp.dot(a1.reshape(TS * K, OUT_F), w2t,
                     preferred_element_type=jnp.float32) + b2v
        h2s.append(h2.reshape(TS, K, OUT_F))
    s1b = jnp.zeros((TS, 1, 1), jnp.float32)
    s2b = jnp.zeros((TS, 1, 1), jnp.float32)
    for b in range(B):
        s1b = s1b + jnp.sum(h2s[b], axis=(1, 2), keepdims=True)
        s2b = s2b + jnp.sum(h2s[b] * h2s[b], axis=(1, 2), keepdims=True)
    meanb = s1b * n_inv
    varb = s2b * n_inv - meanb * meanb
    scb = lax.rsqrt(varb + 1e-5) * gam
    shb = bet - meanb * scb
    for b in range(B):
        a2 = jnp.maximum(h2s[b] * scb + shb, 0.0)
        out_ref[b] = jnp.max(a2, axis=1)


def kernel(x, W1, b1, W2, b2, gamma, beta):
    xflat = x.reshape(B * S, F)
    xt = jnp.transpose(x, (0, 2, 1))
    w1at = jnp.transpose(W1[:, :F])
    w1bt = jnp.transpose(W1[:, F:])
    w2t = jnp.transpose(W2)
    b1r = b1.reshape(1, OUT_F)
    b2r = b2.reshape(1, OUT_F)
    gam2 = gamma.reshape(S, 1)
    bet2 = beta.reshape(S, 1)

    zz, y = pl.pallas_call(
        _zz_kernel,
        out_shape=(
            jax.ShapeDtypeStruct((B * S, OUT_F), jnp.float32),
            jax.ShapeDtypeStruct((B * S, OUT_F), jnp.float32),
        ),
    )(xflat, w1at, w1bt, b1r)

    gs = []
    for b in range(B):
        idx_b = pl.pallas_call(
            functools.partial(_topk_kernel, boff=float(b * S)),
            grid=(1,),
            in_specs=[
                pl.BlockSpec((1, S, F), lambda i, b=b: (b, 0, 0)),
                pl.BlockSpec((1, F, S), lambda i, b=b: (b, 0, 0)),
            ],
            out_specs=pl.BlockSpec((S, K), lambda i: (0, 0)),
            out_shape=jax.ShapeDtypeStruct((S, K), jnp.int32),
            scratch_shapes=[pltpu.VMEM((S, S), jnp.float32)],
        )(x, xt)
        idx3_b = idx_b.reshape(NW, NCH, CH)
        gs.append(_gather_sc(idx3_b, y).reshape(S, K, OUT_F))

    g_specs = [
        pl.BlockSpec((TS, K, OUT_F), lambda i: (i, 0, 0)) for _ in range(B)
    ]
    nzb = (B * S) // (TS * K)
    out = pl.pallas_call(
        _mlp_kernel,
        grid=(S // TS,),
        in_specs=g_specs + [
            pl.BlockSpec((TS * K, OUT_F), lambda i: (i % nzb, 0)),
            pl.BlockSpec((TS, 1), lambda i: (i, 0)),
            pl.BlockSpec((TS, 1), lambda i: (i, 0)),
            pl.BlockSpec((OUT_F, OUT_F), lambda i: (0, 0)),
            pl.BlockSpec((1, OUT_F), lambda i: (0, 0)),
        ],
        out_specs=pl.BlockSpec((B, TS, OUT_F), lambda i: (0, i, 0)),
        out_shape=jax.ShapeDtypeStruct((B, S, OUT_F), jnp.float32),
    )(*gs, zz, gam2, bet2, w2t, b2r)
    return out
